# baseline (speedup 1.0000x reference)
import functools

import jax
import jax.numpy as jnp
from jax import lax
from jax.experimental import pallas as pl
from jax.experimental.pallas import tpu as pltpu
from jax.experimental.pallas import tpu_sc as plsc

NC = 2
NS = 16
LANES = 16
CH = 80
NSLOT = 3
BN = 1000


def _wcomb_block(wc_ref, bp_ref, wbig_ref, R, B, H):
    for r in range(R):
        acc = wc_ref[r, 0] * bp_ref[:, 0:H]
        for b in range(1, B):
            acc = acc + wc_ref[r, b] * bp_ref[:, b * H:(b + 1) * H]
        wbig_ref[:, r * H:(r + 1) * H] = acc


def _project(x, src2, et2, w_comp, basis_p, N, H, RH, R, B, E):
    def body(x_ref, s_ref, e_ref, wc_ref, bp_ref, o_ref, g_ref, wbig_ref):
        _wcomb_block(wc_ref, bp_ref, wbig_ref, R, B, H)
        g_ref[...] = s_ref[...] * R + e_ref[...]
        o_ref[...] = jnp.dot(x_ref[...], wbig_ref[...],
                             preferred_element_type=jnp.float32)

    return pl.pallas_call(
        body,
        grid=(N // BN,),
        in_specs=[pl.BlockSpec((BN, H), lambda i: (i, 0)),
                  pl.BlockSpec((E // 128, 128), lambda i: (0, 0)),
                  pl.BlockSpec((E // 128, 128), lambda i: (0, 0)),
                  pl.BlockSpec(memory_space=pltpu.SMEM),
                  pl.BlockSpec((H, B * H), lambda i: (0, 0))],
        out_specs=[pl.BlockSpec((BN, RH), lambda i: (i, 0)),
                   pl.BlockSpec((E // 128, 128), lambda i: (0, 0))],
        out_shape=[jax.ShapeDtypeStruct((N, RH), jnp.float32),
                   jax.ShapeDtypeStruct((E // 128, 128), jnp.int32)],
        scratch_shapes=[pltpu.VMEM((H, RH), jnp.float32)],
    )(x, src2, et2, w_comp, basis_p)


def _project_fused(parts, bias, w_comp, basis_p, N, H, RH, R, B):
    def body(p_ref, b_ref, wc_ref, bp_ref, o_ref, wbig_ref):
        _wcomb_block(wc_ref, bp_ref, wbig_ref, R, B, H)
        x = jnp.maximum(p_ref[0] + p_ref[1] + b_ref[...], 0.0)
        o_ref[...] = jnp.dot(x, wbig_ref[...],
                             preferred_element_type=jnp.float32)

    return pl.pallas_call(
        body,
        grid=(N // BN,),
        in_specs=[pl.BlockSpec((NC, BN, H), lambda i: (0, i, 0)),
                  pl.BlockSpec((1, H), lambda i: (0, 0)),
                  pl.BlockSpec(memory_space=pltpu.SMEM),
                  pl.BlockSpec((H, B * H), lambda i: (0, 0))],
        out_specs=pl.BlockSpec((BN, RH), lambda i: (i, 0)),
        out_shape=jax.ShapeDtypeStruct((N, RH), jnp.float32),
        scratch_shapes=[pltpu.VMEM((H, RH), jnp.float32)],
    )(parts, bias, w_comp, basis_p)


def _final(parts, bias, N, H):
    def body(p_ref, b_ref, o_ref):
        o_ref[...] = jnp.maximum(p_ref[0] + p_ref[1] + b_ref[...], 0.0)

    return pl.pallas_call(
        body,
        grid=(N // BN,),
        in_specs=[pl.BlockSpec((NC, BN, H), lambda i: (0, i, 0)),
                  pl.BlockSpec((1, H), lambda i: (0, 0))],
        out_specs=pl.BlockSpec((BN, H), lambda i: (i, 0)),
        out_shape=jax.ShapeDtypeStruct((N, H), jnp.float32),
    )(parts, bias)


def _make_edge_kernel(N, H, E_pad, R):
    NT = NC * NS
    EPT = E_pad // NT
    NCH = EPT // CH
    SUP = 5
    C2 = NCH // SUP
    HV = H // LANES
    RPT = (N // NS) & ~7
    RPT_LAST = N - (NS - 1) * RPT
    mesh = plsc.VectorSubcoreMesh(core_axis_name="c", subcore_axis_name="s")

    @functools.partial(
        pl.kernel,
        out_type=jax.ShapeDtypeStruct((NC, N, H), jnp.float32),
        mesh=mesh,
        scratch_types=[
            pltpu.VMEM((C2, CH), jnp.int32),
            pltpu.VMEM((C2, CH), jnp.int32),
            pltpu.VMEM((C2 * CH,), jnp.float32),
            pltpu.VMEM((NSLOT, CH, H), jnp.float32),
            pltpu.VMEM_SHARED((N, H), jnp.float32),
            pltpu.SemaphoreType.DMA((NSLOT,)),
            pltpu.SemaphoreType.DMA((NSLOT,)),
        ],
    )
    def edge_kernel(table, gidx4, dst4, norm3, zeros, out,
                    idx_v, dst_v, norm_v, rows_v, acc, gsem, ssem):
        c = lax.axis_index("c")
        s = lax.axis_index("s")
        wid = c * NS + s
        @pl.when(s < NS - 1)
        def _():
            pltpu.sync_copy(zeros.at[pl.ds(s * RPT, RPT)],
                            acc.at[pl.ds(s * RPT, RPT)])

        @pl.when(s == NS - 1)
        def _():
            pltpu.sync_copy(zeros.at[pl.ds((NS - 1) * RPT, RPT_LAST)],
                            acc.at[pl.ds((NS - 1) * RPT, RPT_LAST)])

        plsc.subcore_barrier()

        def start_gather(j, slot):
            pltpu.async_copy(table.at[idx_v.at[j]], rows_v.at[slot],
                             gsem.at[slot])

        def wait_gather(slot):
            pltpu.make_async_copy(table.at[idx_v.at[0]], rows_v.at[slot],
                                  gsem.at[slot]).wait()

        def start_scatter(j, slot):
            pltpu.async_copy(rows_v.at[slot], acc.at[dst_v.at[j]],
                             ssem.at[slot], add=True)

        def wait_scatter(slot):
            pltpu.make_async_copy(rows_v.at[slot], acc.at[dst_v.at[0]],
                                  ssem.at[slot]).wait()

        def scale(j, slot):
            @plsc.parallel_loop(0, CH // LANES, step=1)
            def grp_body(g):
                nv = norm_v[pl.ds(j * CH + g * LANES, LANES)]
                for t in range(LANES):
                    nb = lax.gather(
                        nv, jnp.full((LANES, 1), t, jnp.int32),
                        dimension_numbers=lax.GatherDimensionNumbers(
                            offset_dims=(), collapsed_slice_dims=(0,),
                            start_index_map=(0,)),
                        slice_sizes=(1,),
                        mode=lax.GatherScatterMode.PROMISE_IN_BOUNDS)
                    e = g * LANES + t
                    for h in range(HV):
                        sl = pl.ds(h * LANES, LANES)
                        rows_v[slot, e, sl] = rows_v[slot, e, sl] * nb

        def sup_body(sup, carry):
            pltpu.sync_copy(gidx4.at[wid, sup], idx_v)
            pltpu.sync_copy(dst4.at[wid, sup], dst_v)
            pltpu.sync_copy(norm3.at[wid, sup], norm_v)
            start_gather(0, 0)

            def step(j, slot):
                nslot = (slot + 1) % NSLOT
                @pl.when(j + 1 < C2)
                def _():
                    @pl.when(j >= NSLOT - 1)
                    def _():
                        wait_scatter(nslot)
                    start_gather(j + 1, nslot)

                wait_gather(slot)
                scale(j, slot)
                start_scatter(j, slot)

            def chunk_body(j, carry2):
                m = lax.rem(j, NSLOT)
                for r in range(NSLOT):
                    @pl.when(m == r)
                    def _(r=r):
                        step(j, r)
                return carry2
            lax.fori_loop(0, C2, chunk_body, 0)
            for r in range(NSLOT):
                wait_scatter(r)
            return carry
        lax.fori_loop(0, SUP, sup_body, 0)

        plsc.subcore_barrier()

        @pl.when(s < NS - 1)
        def _():
            pltpu.sync_copy(acc.at[pl.ds(s * RPT, RPT)],
                            out.at[c, pl.ds(s * RPT, RPT)])

        @pl.when(s == NS - 1)
        def _():
            pltpu.sync_copy(acc.at[pl.ds((NS - 1) * RPT, RPT_LAST)],
                            out.at[c, pl.ds((NS - 1) * RPT, RPT_LAST)])

    return edge_kernel


def kernel(p_feats, edge_index, etype, norm,
           basis0, w_comp0, bias0, basis1, w_comp1, bias1):
    N, H = p_feats.shape
    E = etype.shape[0]
    B = basis0.shape[0]
    R = w_comp0.shape[0]
    RH = R * H
    NT = NC * NS
    SUP = 5
    GRAN = NT * SUP * CH
    E_pad = ((E + GRAN - 1) // GRAN) * GRAN
    pe = E_pad - E
    spread = jnp.arange(pe, dtype=jnp.int32) % N
    src_p = jnp.concatenate([edge_index[0], spread])
    et_p = jnp.concatenate([etype, jnp.zeros((pe,), jnp.int32)])
    dst_p = jnp.concatenate([edge_index[1], spread])
    norm_p = jnp.concatenate([norm.reshape(-1), jnp.zeros((pe,), jnp.float32)])
    EPT = E_pad // NT
    C2 = EPT // (SUP * CH)
    dst4 = dst_p.reshape(NT, SUP, C2, CH)
    norm3 = norm_p.reshape(NT, SUP, C2 * CH)
    zeros = jnp.zeros((N, H), jnp.float32)

    edge_kernel = _make_edge_kernel(N, H, E_pad, R)

    basis_p0 = basis0.transpose(1, 0, 2).reshape(H, B * H)
    xw0, gidx = _project(p_feats, src_p.reshape(E_pad // 128, 128),
                         et_p.reshape(E_pad // 128, 128),
                         w_comp0, basis_p0, N, H, RH, R, B, E_pad)
    gidx4 = gidx.reshape(NT, SUP, C2, CH)
    part0 = edge_kernel(xw0.reshape(N * R, H), gidx4, dst4, norm3, zeros)

    basis_p1 = basis1.transpose(1, 0, 2).reshape(H, B * H)
    xw1 = _project_fused(part0, bias0.reshape(1, H), w_comp1, basis_p1,
                         N, H, RH, R, B)
    part1 = edge_kernel(xw1.reshape(N * R, H), gidx4, dst4, norm3, zeros)

    return _final(part1, bias1.reshape(1, H), N, H)


# speedup vs baseline: 1.0564x; 1.0564x over previous
import functools

import jax
import jax.numpy as jnp
from jax import lax
from jax.experimental import pallas as pl
from jax.experimental.pallas import tpu as pltpu
from jax.experimental.pallas import tpu_sc as plsc

NC = 2
NS = 16
LANES = 16
CH = 128
NSLOT = 2
BN = 1000


def _wcomb_block(wc_ref, bp_ref, wbig_ref, R, B, H):
    for r in range(R):
        acc = wc_ref[r, 0] * bp_ref[:, 0:H]
        for b in range(1, B):
            acc = acc + wc_ref[r, b] * bp_ref[:, b * H:(b + 1) * H]
        wbig_ref[:, r * H:(r + 1) * H] = acc


def _project(x, src2, et2, w_comp, basis_p, N, H, RH, R, B, E):
    def body(x_ref, s_ref, e_ref, wc_ref, bp_ref, o_ref, g_ref, wbig_ref):
        _wcomb_block(wc_ref, bp_ref, wbig_ref, R, B, H)
        g_ref[...] = s_ref[...] * R + e_ref[...]
        o_ref[...] = jnp.dot(x_ref[...], wbig_ref[...],
                             preferred_element_type=jnp.float32)

    return pl.pallas_call(
        body,
        grid=(N // BN,),
        in_specs=[pl.BlockSpec((BN, H), lambda i: (i, 0)),
                  pl.BlockSpec((E // 128, 128), lambda i: (0, 0)),
                  pl.BlockSpec((E // 128, 128), lambda i: (0, 0)),
                  pl.BlockSpec(memory_space=pltpu.SMEM),
                  pl.BlockSpec((H, B * H), lambda i: (0, 0))],
        out_specs=[pl.BlockSpec((BN, RH), lambda i: (i, 0)),
                   pl.BlockSpec((E // 128, 128), lambda i: (0, 0))],
        out_shape=[jax.ShapeDtypeStruct((N, RH), jnp.float32),
                   jax.ShapeDtypeStruct((E // 128, 128), jnp.int32)],
        scratch_shapes=[pltpu.VMEM((H, RH), jnp.float32)],
    )(x, src2, et2, w_comp, basis_p)


def _project_fused(parts, bias, w_comp, basis_p, N, H, RH, R, B):
    def body(p_ref, b_ref, wc_ref, bp_ref, o_ref, wbig_ref):
        _wcomb_block(wc_ref, bp_ref, wbig_ref, R, B, H)
        x = jnp.maximum(p_ref[0] + p_ref[1] + b_ref[...], 0.0)
        o_ref[...] = jnp.dot(x, wbig_ref[...],
                             preferred_element_type=jnp.float32)

    return pl.pallas_call(
        body,
        grid=(N // BN,),
        in_specs=[pl.BlockSpec((NC, BN, H), lambda i: (0, i, 0)),
                  pl.BlockSpec((1, H), lambda i: (0, 0)),
                  pl.BlockSpec(memory_space=pltpu.SMEM),
                  pl.BlockSpec((H, B * H), lambda i: (0, 0))],
        out_specs=pl.BlockSpec((BN, RH), lambda i: (i, 0)),
        out_shape=jax.ShapeDtypeStruct((N, RH), jnp.float32),
        scratch_shapes=[pltpu.VMEM((H, RH), jnp.float32)],
    )(parts, bias, w_comp, basis_p)


def _final(parts, bias, N, H):
    def body(p_ref, b_ref, o_ref):
        o_ref[...] = jnp.maximum(p_ref[0] + p_ref[1] + b_ref[...], 0.0)

    return pl.pallas_call(
        body,
        grid=(N // BN,),
        in_specs=[pl.BlockSpec((NC, BN, H), lambda i: (0, i, 0)),
                  pl.BlockSpec((1, H), lambda i: (0, 0))],
        out_specs=pl.BlockSpec((BN, H), lambda i: (i, 0)),
        out_shape=jax.ShapeDtypeStruct((N, H), jnp.float32),
    )(parts, bias)


def _make_edge_kernel(N, H, E_pad, R):
    NT = NC * NS
    EPT = E_pad // NT
    NCH = EPT // CH
    SUP = 5
    C2 = NCH // SUP
    HV = H // LANES
    RPT = (N // NS) & ~7
    RPT_LAST = N - (NS - 1) * RPT
    mesh = plsc.VectorSubcoreMesh(core_axis_name="c", subcore_axis_name="s")

    @functools.partial(
        pl.kernel,
        out_type=jax.ShapeDtypeStruct((NC, N, H), jnp.float32),
        mesh=mesh,
        scratch_types=[
            pltpu.VMEM((C2, CH), jnp.int32),
            pltpu.VMEM((C2, CH), jnp.int32),
            pltpu.VMEM((C2 * CH,), jnp.float32),
            pltpu.VMEM((NSLOT, CH, H), jnp.float32),
            pltpu.VMEM_SHARED((N, H), jnp.float32),
            pltpu.SemaphoreType.DMA((NSLOT,)),
            pltpu.SemaphoreType.DMA((NSLOT,)),
        ],
    )
    def edge_kernel(table, gidx4, dst4, norm3, zeros, out,
                    idx_v, dst_v, norm_v, rows_v, acc, gsem, ssem):
        c = lax.axis_index("c")
        s = lax.axis_index("s")
        wid = c * NS + s
        @pl.when(s < NS - 1)
        def _():
            pltpu.sync_copy(zeros.at[pl.ds(s * RPT, RPT)],
                            acc.at[pl.ds(s * RPT, RPT)])

        @pl.when(s == NS - 1)
        def _():
            pltpu.sync_copy(zeros.at[pl.ds((NS - 1) * RPT, RPT_LAST)],
                            acc.at[pl.ds((NS - 1) * RPT, RPT_LAST)])

        plsc.subcore_barrier()

        def start_gather(j, slot):
            pltpu.async_copy(table.at[idx_v.at[j]], rows_v.at[slot],
                             gsem.at[slot])

        def wait_gather(slot):
            pltpu.make_async_copy(table.at[idx_v.at[0]], rows_v.at[slot],
                                  gsem.at[slot]).wait()

        def start_scatter(j, slot):
            pltpu.async_copy(rows_v.at[slot], acc.at[dst_v.at[j]],
                             ssem.at[slot], add=True)

        def wait_scatter(slot):
            pltpu.make_async_copy(rows_v.at[slot], acc.at[dst_v.at[0]],
                                  ssem.at[slot]).wait()

        def scale(j, slot):
            @plsc.parallel_loop(0, CH // LANES, step=1)
            def grp_body(g):
                nv = norm_v[pl.ds(j * CH + g * LANES, LANES)]
                for t in range(LANES):
                    nb = lax.gather(
                        nv, jnp.full((LANES, 1), t, jnp.int32),
                        dimension_numbers=lax.GatherDimensionNumbers(
                            offset_dims=(), collapsed_slice_dims=(0,),
                            start_index_map=(0,)),
                        slice_sizes=(1,),
                        mode=lax.GatherScatterMode.PROMISE_IN_BOUNDS)
                    e = g * LANES + t
                    for h in range(HV):
                        sl = pl.ds(h * LANES, LANES)
                        rows_v[slot, e, sl] = rows_v[slot, e, sl] * nb

        def sup_body(sup, carry):
            pltpu.sync_copy(gidx4.at[wid, sup], idx_v)
            pltpu.sync_copy(dst4.at[wid, sup], dst_v)
            pltpu.sync_copy(norm3.at[wid, sup], norm_v)
            start_gather(0, 0)

            def step(j, slot):
                nslot = (slot + 1) % NSLOT
                @pl.when(j + 1 < C2)
                def _():
                    @pl.when(j >= NSLOT - 1)
                    def _():
                        wait_scatter(nslot)
                    start_gather(j + 1, nslot)

                wait_gather(slot)
                scale(j, slot)
                start_scatter(j, slot)

            def chunk_body(j, carry2):
                m = lax.rem(j, NSLOT)
                for r in range(NSLOT):
                    @pl.when(m == r)
                    def _(r=r):
                        step(j, r)
                return carry2
            lax.fori_loop(0, C2, chunk_body, 0)
            for r in range(NSLOT):
                wait_scatter(r)
            return carry
        lax.fori_loop(0, SUP, sup_body, 0)

        plsc.subcore_barrier()

        @pl.when(s < NS - 1)
        def _():
            pltpu.sync_copy(acc.at[pl.ds(s * RPT, RPT)],
                            out.at[c, pl.ds(s * RPT, RPT)])

        @pl.when(s == NS - 1)
        def _():
            pltpu.sync_copy(acc.at[pl.ds((NS - 1) * RPT, RPT_LAST)],
                            out.at[c, pl.ds((NS - 1) * RPT, RPT_LAST)])

    return edge_kernel


def kernel(p_feats, edge_index, etype, norm,
           basis0, w_comp0, bias0, basis1, w_comp1, bias1):
    N, H = p_feats.shape
    E = etype.shape[0]
    B = basis0.shape[0]
    R = w_comp0.shape[0]
    RH = R * H
    NT = NC * NS
    SUP = 5
    GRAN = NT * SUP * CH
    E_pad = ((E + GRAN - 1) // GRAN) * GRAN
    pe = E_pad - E
    spread = jnp.arange(pe, dtype=jnp.int32) % N
    src_p = jnp.concatenate([edge_index[0], spread])
    et_p = jnp.concatenate([etype, jnp.zeros((pe,), jnp.int32)])
    dst_p = jnp.concatenate([edge_index[1], spread])
    norm_p = jnp.concatenate([norm.reshape(-1), jnp.zeros((pe,), jnp.float32)])
    EPT = E_pad // NT
    C2 = EPT // (SUP * CH)
    dst4 = dst_p.reshape(NT, SUP, C2, CH)
    norm3 = norm_p.reshape(NT, SUP, C2 * CH)
    zeros = jnp.zeros((N, H), jnp.float32)

    edge_kernel = _make_edge_kernel(N, H, E_pad, R)

    basis_p0 = basis0.transpose(1, 0, 2).reshape(H, B * H)
    xw0, gidx = _project(p_feats, src_p.reshape(E_pad // 128, 128),
                         et_p.reshape(E_pad // 128, 128),
                         w_comp0, basis_p0, N, H, RH, R, B, E_pad)
    gidx4 = gidx.reshape(NT, SUP, C2, CH)
    part0 = edge_kernel(xw0.reshape(N * R, H), gidx4, dst4, norm3, zeros)

    basis_p1 = basis1.transpose(1, 0, 2).reshape(H, B * H)
    xw1 = _project_fused(part0, bias0.reshape(1, H), w_comp1, basis_p1,
                         N, H, RH, R, B)
    part1 = edge_kernel(xw1.reshape(N * R, H), gidx4, dst4, norm3, zeros)

    return _final(part1, bias1.reshape(1, H), N, H)


# speedup vs baseline: 1.1325x; 1.0721x over previous
import functools

import jax
import jax.numpy as jnp
from jax import lax
from jax.experimental import pallas as pl
from jax.experimental.pallas import tpu as pltpu
from jax.experimental.pallas import tpu_sc as plsc

NC = 2
NS = 16
LANES = 16
CH = 128
NSLOT = 2
BN = 1000


def _wcomb_block(wc_ref, bp_ref, wbig_ref, R, B, H):
    for r in range(R):
        acc = wc_ref[r, 0] * bp_ref[:, 0:H]
        for b in range(1, B):
            acc = acc + wc_ref[r, b] * bp_ref[:, b * H:(b + 1) * H]
        wbig_ref[:, r * H:(r + 1) * H] = acc


def _project(x, src2, et2, w_comp, basis_p, N, H, RH, R, B, E):
    def body(x_ref, s_ref, e_ref, wc_ref, bp_ref, o_ref, g_ref):
        i = pl.program_id(0)
        r = pl.program_id(1)

        @pl.when(jnp.logical_and(i == 0, r == 0))
        def _():
            g_ref[...] = e_ref[...] * N + s_ref[...]

        wr = wc_ref[r, 0] * bp_ref[:, 0:H]
        for b in range(1, B):
            wr = wr + wc_ref[r, b] * bp_ref[:, b * H:(b + 1) * H]
        o_ref[0] = jnp.dot(x_ref[...], wr, preferred_element_type=jnp.float32)

    return pl.pallas_call(
        body,
        grid=(N // BN, R),
        in_specs=[pl.BlockSpec((BN, H), lambda i, r: (i, 0)),
                  pl.BlockSpec((E // 128, 128), lambda i, r: (0, 0)),
                  pl.BlockSpec((E // 128, 128), lambda i, r: (0, 0)),
                  pl.BlockSpec(memory_space=pltpu.SMEM),
                  pl.BlockSpec((H, B * H), lambda i, r: (0, 0))],
        out_specs=[pl.BlockSpec((1, BN, H), lambda i, r: (r, i, 0)),
                   pl.BlockSpec((E // 128, 128), lambda i, r: (0, 0))],
        out_shape=[jax.ShapeDtypeStruct((R, N, H), jnp.float32),
                   jax.ShapeDtypeStruct((E // 128, 128), jnp.int32)],
    )(x, src2, et2, w_comp, basis_p)


def _project_fused(parts, bias, w_comp, basis_p, N, H, RH, R, B):
    def body(p_ref, b_ref, wc_ref, bp_ref, o_ref):
        r = pl.program_id(1)
        x = jnp.maximum(p_ref[0] + p_ref[1] + b_ref[...], 0.0)
        wr = wc_ref[r, 0] * bp_ref[:, 0:H]
        for b in range(1, B):
            wr = wr + wc_ref[r, b] * bp_ref[:, b * H:(b + 1) * H]
        o_ref[0] = jnp.dot(x, wr, preferred_element_type=jnp.float32)

    return pl.pallas_call(
        body,
        grid=(N // BN, R),
        in_specs=[pl.BlockSpec((NC, BN, H), lambda i, r: (0, i, 0)),
                  pl.BlockSpec((1, H), lambda i, r: (0, 0)),
                  pl.BlockSpec(memory_space=pltpu.SMEM),
                  pl.BlockSpec((H, B * H), lambda i, r: (0, 0))],
        out_specs=pl.BlockSpec((1, BN, H), lambda i, r: (r, i, 0)),
        out_shape=jax.ShapeDtypeStruct((R, N, H), jnp.float32),
    )(parts, bias, w_comp, basis_p)


def _final(parts, bias, N, H):
    def body(p_ref, b_ref, o_ref):
        o_ref[...] = jnp.maximum(p_ref[0] + p_ref[1] + b_ref[...], 0.0)

    return pl.pallas_call(
        body,
        grid=(N // BN,),
        in_specs=[pl.BlockSpec((NC, BN, H), lambda i: (0, i, 0)),
                  pl.BlockSpec((1, H), lambda i: (0, 0))],
        out_specs=pl.BlockSpec((BN, H), lambda i: (i, 0)),
        out_shape=jax.ShapeDtypeStruct((N, H), jnp.float32),
    )(parts, bias)


def _make_edge_kernel(N, H, E_pad, R):
    NT = NC * NS
    EPT = E_pad // NT
    NCH = EPT // CH
    SUP = 5
    C2 = NCH // SUP
    HV = H // LANES
    RPT = (N // NS) & ~7
    RPT_LAST = N - (NS - 1) * RPT
    mesh = plsc.VectorSubcoreMesh(core_axis_name="c", subcore_axis_name="s")

    @functools.partial(
        pl.kernel,
        out_type=jax.ShapeDtypeStruct((NC, N, H), jnp.float32),
        mesh=mesh,
        scratch_types=[
            pltpu.VMEM((C2, CH), jnp.int32),
            pltpu.VMEM((C2, CH), jnp.int32),
            pltpu.VMEM((C2 * CH,), jnp.float32),
            pltpu.VMEM((NSLOT, CH, H), jnp.float32),
            pltpu.VMEM_SHARED((N, H), jnp.float32),
            pltpu.SemaphoreType.DMA((NSLOT,)),
            pltpu.SemaphoreType.DMA((NSLOT,)),
        ],
    )
    def edge_kernel(table, gidx4, dst4, norm3, zeros, out,
                    idx_v, dst_v, norm_v, rows_v, acc, gsem, ssem):
        c = lax.axis_index("c")
        s = lax.axis_index("s")
        wid = c * NS + s
        @pl.when(s < NS - 1)
        def _():
            pltpu.sync_copy(zeros.at[pl.ds(s * RPT, RPT)],
                            acc.at[pl.ds(s * RPT, RPT)])

        @pl.when(s == NS - 1)
        def _():
            pltpu.sync_copy(zeros.at[pl.ds((NS - 1) * RPT, RPT_LAST)],
                            acc.at[pl.ds((NS - 1) * RPT, RPT_LAST)])

        plsc.subcore_barrier()

        def start_gather(j, slot):
            pltpu.async_copy(table.at[idx_v.at[j]], rows_v.at[slot],
                             gsem.at[slot])

        def wait_gather(slot):
            pltpu.make_async_copy(table.at[idx_v.at[0]], rows_v.at[slot],
                                  gsem.at[slot]).wait()

        def start_scatter(j, slot):
            pltpu.async_copy(rows_v.at[slot], acc.at[dst_v.at[j]],
                             ssem.at[slot], add=True)

        def wait_scatter(slot):
            pltpu.make_async_copy(rows_v.at[slot], acc.at[dst_v.at[0]],
                                  ssem.at[slot]).wait()

        def scale(j, slot):
            @plsc.parallel_loop(0, CH // LANES, step=1)
            def grp_body(g):
                nv = norm_v[pl.ds(j * CH + g * LANES, LANES)]
                for t in range(LANES):
                    nb = lax.gather(
                        nv, jnp.full((LANES, 1), t, jnp.int32),
                        dimension_numbers=lax.GatherDimensionNumbers(
                            offset_dims=(), collapsed_slice_dims=(0,),
                            start_index_map=(0,)),
                        slice_sizes=(1,),
                        mode=lax.GatherScatterMode.PROMISE_IN_BOUNDS)
                    e = g * LANES + t
                    for h in range(HV):
                        sl = pl.ds(h * LANES, LANES)
                        rows_v[slot, e, sl] = rows_v[slot, e, sl] * nb

        def sup_body(sup, carry):
            pltpu.sync_copy(gidx4.at[wid, sup], idx_v)
            pltpu.sync_copy(dst4.at[wid, sup], dst_v)
            pltpu.sync_copy(norm3.at[wid, sup], norm_v)
            start_gather(0, 0)

            def step(j, slot):
                nslot = (slot + 1) % NSLOT
                @pl.when(j + 1 < C2)
                def _():
                    @pl.when(j >= NSLOT - 1)
                    def _():
                        wait_scatter(nslot)
                    start_gather(j + 1, nslot)

                wait_gather(slot)
                scale(j, slot)
                start_scatter(j, slot)

            def chunk_body(j, carry2):
                m = lax.rem(j, NSLOT)
                for r in range(NSLOT):
                    @pl.when(m == r)
                    def _(r=r):
                        step(j, r)
                return carry2
            lax.fori_loop(0, C2, chunk_body, 0)
            for r in range(NSLOT):
                wait_scatter(r)
            return carry
        lax.fori_loop(0, SUP, sup_body, 0)

        plsc.subcore_barrier()

        @pl.when(s < NS - 1)
        def _():
            pltpu.sync_copy(acc.at[pl.ds(s * RPT, RPT)],
                            out.at[c, pl.ds(s * RPT, RPT)])

        @pl.when(s == NS - 1)
        def _():
            pltpu.sync_copy(acc.at[pl.ds((NS - 1) * RPT, RPT_LAST)],
                            out.at[c, pl.ds((NS - 1) * RPT, RPT_LAST)])

    return edge_kernel


def kernel(p_feats, edge_index, etype, norm,
           basis0, w_comp0, bias0, basis1, w_comp1, bias1):
    N, H = p_feats.shape
    E = etype.shape[0]
    B = basis0.shape[0]
    R = w_comp0.shape[0]
    RH = R * H
    NT = NC * NS
    SUP = 5
    GRAN = NT * SUP * CH
    E_pad = ((E + GRAN - 1) // GRAN) * GRAN
    pe = E_pad - E
    spread = jnp.arange(pe, dtype=jnp.int32) % N
    src_p = jnp.concatenate([edge_index[0], spread])
    et_p = jnp.concatenate([etype, jnp.zeros((pe,), jnp.int32)])
    dst_p = jnp.concatenate([edge_index[1], spread])
    norm_p = jnp.concatenate([norm.reshape(-1), jnp.zeros((pe,), jnp.float32)])
    EPT = E_pad // NT
    C2 = EPT // (SUP * CH)
    dst4 = dst_p.reshape(NT, SUP, C2, CH)
    norm3 = norm_p.reshape(NT, SUP, C2 * CH)
    zeros = jnp.zeros((N, H), jnp.float32)

    edge_kernel = _make_edge_kernel(N, H, E_pad, R)

    basis_p0 = basis0.transpose(1, 0, 2).reshape(H, B * H)
    xw0, gidx = _project(p_feats, src_p.reshape(E_pad // 128, 128),
                         et_p.reshape(E_pad // 128, 128),
                         w_comp0, basis_p0, N, H, RH, R, B, E_pad)
    gidx4 = gidx.reshape(NT, SUP, C2, CH)
    part0 = edge_kernel(xw0.reshape(R * N, H), gidx4, dst4, norm3, zeros)

    basis_p1 = basis1.transpose(1, 0, 2).reshape(H, B * H)
    xw1 = _project_fused(part0, bias0.reshape(1, H), w_comp1, basis_p1,
                         N, H, RH, R, B)
    part1 = edge_kernel(xw1.reshape(R * N, H), gidx4, dst4, norm3, zeros)

    return _final(part1, bias1.reshape(1, H), N, H)


# speedup vs baseline: 1.4983x; 1.3230x over previous
import functools

import jax
import jax.numpy as jnp
from jax import lax
from jax.experimental import pallas as pl
from jax.experimental.pallas import tpu as pltpu
from jax.experimental.pallas import tpu_sc as plsc

NC = 2
NS = 16
LANES = 16
CH = 128
NSLOT = 2
BN = 1000


def _wcomb_block(wc_ref, bp_ref, wbig_ref, R, B, H):
    for r in range(R):
        acc = wc_ref[r, 0] * bp_ref[:, 0:H]
        for b in range(1, B):
            acc = acc + wc_ref[r, b] * bp_ref[:, b * H:(b + 1) * H]
        wbig_ref[:, r * H:(r + 1) * H] = acc


def _project(x, src2, et2, w_comp, basis_p, N, H, RH, R, B, E):
    def body(x_ref, s_ref, e_ref, wc_ref, bp_ref, o_ref, g_ref, wbig_ref):
        @pl.when(pl.program_id(0) == 0)
        def _():
            g_ref[...] = e_ref[...] * N + s_ref[...]

        _wcomb_block(wc_ref, bp_ref, wbig_ref, R, B, H)
        xw = jnp.dot(x_ref[...], wbig_ref[...],
                     preferred_element_type=jnp.float32)
        for r in range(R):
            o_ref[r] = xw[:, r * H:(r + 1) * H]

    return pl.pallas_call(
        body,
        grid=(N // BN,),
        in_specs=[pl.BlockSpec((BN, H), lambda i: (i, 0)),
                  pl.BlockSpec((E // 128, 128), lambda i: (0, 0)),
                  pl.BlockSpec((E // 128, 128), lambda i: (0, 0)),
                  pl.BlockSpec(memory_space=pltpu.SMEM),
                  pl.BlockSpec((H, B * H), lambda i: (0, 0))],
        out_specs=[pl.BlockSpec((R, BN, H), lambda i: (0, i, 0)),
                   pl.BlockSpec((E // 128, 128), lambda i: (0, 0))],
        out_shape=[jax.ShapeDtypeStruct((R, N, H), jnp.float32),
                   jax.ShapeDtypeStruct((E // 128, 128), jnp.int32)],
        scratch_shapes=[pltpu.VMEM((H, RH), jnp.float32)],
    )(x, src2, et2, w_comp, basis_p)


def _project_fused(parts, bias, w_comp, basis_p, N, H, RH, R, B):
    def body(p_ref, b_ref, wc_ref, bp_ref, o_ref, wbig_ref):
        _wcomb_block(wc_ref, bp_ref, wbig_ref, R, B, H)
        x = jnp.maximum(p_ref[0] + p_ref[1] + b_ref[...], 0.0)
        xw = jnp.dot(x, wbig_ref[...], preferred_element_type=jnp.float32)
        for r in range(R):
            o_ref[r] = xw[:, r * H:(r + 1) * H]

    return pl.pallas_call(
        body,
        grid=(N // BN,),
        in_specs=[pl.BlockSpec((NC, BN, H), lambda i: (0, i, 0)),
                  pl.BlockSpec((1, H), lambda i: (0, 0)),
                  pl.BlockSpec(memory_space=pltpu.SMEM),
                  pl.BlockSpec((H, B * H), lambda i: (0, 0))],
        out_specs=pl.BlockSpec((R, BN, H), lambda i: (0, i, 0)),
        out_shape=jax.ShapeDtypeStruct((R, N, H), jnp.float32),
        scratch_shapes=[pltpu.VMEM((H, RH), jnp.float32)],
    )(parts, bias, w_comp, basis_p)


def _final(parts, bias, N, H):
    def body(p_ref, b_ref, o_ref):
        o_ref[...] = jnp.maximum(p_ref[0] + p_ref[1] + b_ref[...], 0.0)

    return pl.pallas_call(
        body,
        grid=(N // BN,),
        in_specs=[pl.BlockSpec((NC, BN, H), lambda i: (0, i, 0)),
                  pl.BlockSpec((1, H), lambda i: (0, 0))],
        out_specs=pl.BlockSpec((BN, H), lambda i: (i, 0)),
        out_shape=jax.ShapeDtypeStruct((N, H), jnp.float32),
    )(parts, bias)


def _make_edge_kernel(N, H, E_pad, R):
    NT = NC * NS
    EPT = E_pad // NT
    NCH = EPT // CH
    SUP = 5
    C2 = NCH // SUP
    HV = H // LANES
    RPT = (N // NS) & ~7
    RPT_LAST = N - (NS - 1) * RPT
    mesh = plsc.VectorSubcoreMesh(core_axis_name="c", subcore_axis_name="s")

    @functools.partial(
        pl.kernel,
        out_type=jax.ShapeDtypeStruct((NC, N, H), jnp.float32),
        mesh=mesh,
        scratch_types=[
            pltpu.VMEM((C2, CH), jnp.int32),
            pltpu.VMEM((C2, CH), jnp.int32),
            pltpu.VMEM((C2 * CH,), jnp.float32),
            pltpu.VMEM((NSLOT, CH, H), jnp.float32),
            pltpu.VMEM_SHARED((N, H), jnp.float32),
            pltpu.SemaphoreType.DMA((NSLOT,)),
            pltpu.SemaphoreType.DMA((NSLOT,)),
        ],
    )
    def edge_kernel(table, gidx4, dst4, norm3, zeros, out,
                    idx_v, dst_v, norm_v, rows_v, acc, gsem, ssem):
        c = lax.axis_index("c")
        s = lax.axis_index("s")
        wid = c * NS + s
        @pl.when(s < NS - 1)
        def _():
            pltpu.sync_copy(zeros.at[pl.ds(s * RPT, RPT)],
                            acc.at[pl.ds(s * RPT, RPT)])

        @pl.when(s == NS - 1)
        def _():
            pltpu.sync_copy(zeros.at[pl.ds((NS - 1) * RPT, RPT_LAST)],
                            acc.at[pl.ds((NS - 1) * RPT, RPT_LAST)])

        plsc.subcore_barrier()

        def start_gather(j, slot):
            pltpu.async_copy(table.at[idx_v.at[j]], rows_v.at[slot],
                             gsem.at[slot])

        def wait_gather(slot):
            pltpu.make_async_copy(table.at[idx_v.at[0]], rows_v.at[slot],
                                  gsem.at[slot]).wait()

        def start_scatter(j, slot):
            pltpu.async_copy(rows_v.at[slot], acc.at[dst_v.at[j]],
                             ssem.at[slot], add=True)

        def wait_scatter(slot):
            pltpu.make_async_copy(rows_v.at[slot], acc.at[dst_v.at[0]],
                                  ssem.at[slot]).wait()

        def scale(j, slot):
            @plsc.parallel_loop(0, CH // LANES, step=1)
            def grp_body(g):
                nv = norm_v[pl.ds(j * CH + g * LANES, LANES)]
                for t in range(LANES):
                    nb = lax.gather(
                        nv, jnp.full((LANES, 1), t, jnp.int32),
                        dimension_numbers=lax.GatherDimensionNumbers(
                            offset_dims=(), collapsed_slice_dims=(0,),
                            start_index_map=(0,)),
                        slice_sizes=(1,),
                        mode=lax.GatherScatterMode.PROMISE_IN_BOUNDS)
                    e = g * LANES + t
                    for h in range(HV):
                        sl = pl.ds(h * LANES, LANES)
                        rows_v[slot, e, sl] = rows_v[slot, e, sl] * nb

        def sup_body(sup, carry):
            pltpu.sync_copy(gidx4.at[wid, sup], idx_v)
            pltpu.sync_copy(dst4.at[wid, sup], dst_v)
            pltpu.sync_copy(norm3.at[wid, sup], norm_v)
            start_gather(0, 0)

            def step(j, slot):
                nslot = (slot + 1) % NSLOT
                @pl.when(j + 1 < C2)
                def _():
                    @pl.when(j >= NSLOT - 1)
                    def _():
                        wait_scatter(nslot)
                    start_gather(j + 1, nslot)

                wait_gather(slot)
                scale(j, slot)
                start_scatter(j, slot)

            def chunk_body(j, carry2):
                m = lax.rem(j, NSLOT)
                for r in range(NSLOT):
                    @pl.when(m == r)
                    def _(r=r):
                        step(j, r)
                return carry2
            lax.fori_loop(0, C2, chunk_body, 0)
            for r in range(NSLOT):
                wait_scatter(r)
            return carry
        lax.fori_loop(0, SUP, sup_body, 0)

        plsc.subcore_barrier()

        @pl.when(s < NS - 1)
        def _():
            pltpu.sync_copy(acc.at[pl.ds(s * RPT, RPT)],
                            out.at[c, pl.ds(s * RPT, RPT)])

        @pl.when(s == NS - 1)
        def _():
            pltpu.sync_copy(acc.at[pl.ds((NS - 1) * RPT, RPT_LAST)],
                            out.at[c, pl.ds((NS - 1) * RPT, RPT_LAST)])

    return edge_kernel


def kernel(p_feats, edge_index, etype, norm,
           basis0, w_comp0, bias0, basis1, w_comp1, bias1):
    N, H = p_feats.shape
    E = etype.shape[0]
    B = basis0.shape[0]
    R = w_comp0.shape[0]
    RH = R * H
    NT = NC * NS
    SUP = 5
    GRAN = NT * SUP * CH
    E_pad = ((E + GRAN - 1) // GRAN) * GRAN
    pe = E_pad - E
    spread = jnp.arange(pe, dtype=jnp.int32) % N
    src_p = jnp.concatenate([edge_index[0], spread])
    et_p = jnp.concatenate([etype, jnp.zeros((pe,), jnp.int32)])
    dst_p = jnp.concatenate([edge_index[1], spread])
    norm_p = jnp.concatenate([norm.reshape(-1), jnp.zeros((pe,), jnp.float32)])
    EPT = E_pad // NT
    C2 = EPT // (SUP * CH)
    dst4 = dst_p.reshape(NT, SUP, C2, CH)
    norm3 = norm_p.reshape(NT, SUP, C2 * CH)
    zeros = jnp.zeros((N, H), jnp.float32)

    edge_kernel = _make_edge_kernel(N, H, E_pad, R)

    basis_p0 = basis0.transpose(1, 0, 2).reshape(H, B * H)
    xw0, gidx = _project(p_feats, src_p.reshape(E_pad // 128, 128),
                         et_p.reshape(E_pad // 128, 128),
                         w_comp0, basis_p0, N, H, RH, R, B, E_pad)
    gidx4 = gidx.reshape(NT, SUP, C2, CH)
    part0 = edge_kernel(xw0.reshape(R * N, H), gidx4, dst4, norm3, zeros)

    basis_p1 = basis1.transpose(1, 0, 2).reshape(H, B * H)
    xw1 = _project_fused(part0, bias0.reshape(1, H), w_comp1, basis_p1,
                         N, H, RH, R, B)
    part1 = edge_kernel(xw1.reshape(R * N, H), gidx4, dst4, norm3, zeros)

    return _final(part1, bias1.reshape(1, H), N, H)
